# Initial kernel scaffold; baseline (speedup 1.0000x reference)
import jax
import jax.numpy as jnp
from jax.experimental import pallas as pl


def kernel(x, edge_index, W1, a_src1, a_dst1, b1, W2, a_src2, a_dst2, b2, fc_W, fc_b):
    raise NotImplementedError("write your pallas kernel here")


# speedup vs baseline: 1.0000x; 1.0000x over previous
import jax
import jax.numpy as jnp
from jax.experimental import pallas as pl


def _gat_conv(x, edge_index, W, a_src, a_dst, b, heads, out_ch, concat):
    N = x.shape[0]
    loop = jnp.arange(N, dtype=edge_index.dtype)
    src = jnp.concatenate([edge_index[0], loop])
    dst = jnp.concatenate([edge_index[1], loop])
    h = (x @ W).reshape(N, heads, out_ch)
    alpha_src = (h * a_src).sum(-1)
    alpha_dst = (h * a_dst).sum(-1)
    alpha = alpha_src[src] + alpha_dst[dst]
    alpha = jax.nn.leaky_relu(alpha, negative_slope=0.2)
    amax = jax.ops.segment_max(alpha, dst, num_segments=N)
    alpha = jnp.exp(alpha - amax[dst])
    denom = jax.ops.segment_sum(alpha, dst, num_segments=N)
    alpha = alpha / (denom[dst] + 1e-16)
    msg = h[src] * alpha[:, :, None]
    out = jax.ops.segment_sum(msg, dst, num_segments=N)
    if concat:
        out = out.reshape(N, heads * out_ch)
    else:
        out = out.mean(axis=1)
    return out + b


def _bias_kernel(h_ref, w_ref, b_ref, o_ref):
    o_ref[...] = h_ref[...] @ w_ref[...] + b_ref[...]


def kernel(x, edge_index, W1, a_src1, a_dst1, b1, W2, a_src2, a_dst2, b2, fc_W, fc_b):
    h = jax.nn.relu(_gat_conv(x, edge_index, W1, a_src1, a_dst1, b1, heads=8, out_ch=64, concat=True))
    h = jax.nn.relu(_gat_conv(h, edge_index, W2, a_src2, a_dst2, b2, heads=1, out_ch=32, concat=False))
    embedding = h
    out = pl.pallas_call(
        _bias_kernel,
        out_shape=jax.ShapeDtypeStruct((h.shape[0], fc_W.shape[1]), h.dtype),
    )(h, fc_W, fc_b[None, :])
    return (embedding, out)


# speedup vs baseline: 27.5635x; 27.5629x over previous
import functools

import jax
import jax.numpy as jnp
from jax import lax
from jax.experimental import pallas as pl
from jax.experimental.pallas import tpu as pltpu
from jax.experimental.pallas import tpu_sc as plsc

N = 10000
E = 320000
DIN = 128
H1 = 8
C1 = 64
D1 = H1 * C1
C2 = 32
DOUT = 64

NC = 2
NS = 16
NW = NC * NS


_BN1 = 1000


def _tc1_body(x_ref, w1_ref, asd_ref, h1t_ref, saA_ref, saB_ref):
    h = jnp.dot(x_ref[...], w1_ref[...], preferred_element_type=jnp.float32,
                 precision=lax.Precision.HIGHEST)
    sa = jnp.dot(h, asd_ref[...], preferred_element_type=jnp.float32,
                 precision=lax.Precision.HIGHEST)
    saA_ref[...] = sa[:, :H1]
    saB_ref[...] = sa[:, H1:]
    for hd in range(H1):
        h1t_ref[hd] = h[:, hd * C1:(hd + 1) * C1]


def _tc1(x, W1, Asd1):
    grid = (N // _BN1,)
    return pl.pallas_call(
        _tc1_body,
        grid=grid,
        in_specs=[
            pl.BlockSpec((_BN1, DIN), lambda i: (i, 0)),
            pl.BlockSpec((DIN, D1), lambda i: (0, 0)),
            pl.BlockSpec((D1, 2 * H1), lambda i: (0, 0)),
        ],
        out_specs=[
            pl.BlockSpec((H1, _BN1, C1), lambda i: (0, i, 0)),
            pl.BlockSpec((_BN1, H1), lambda i: (i, 0)),
            pl.BlockSpec((_BN1, H1), lambda i: (i, 0)),
        ],
        out_shape=[
            jax.ShapeDtypeStruct((H1, N, C1), jnp.float32),
            jax.ShapeDtypeStruct((N, H1), jnp.float32),
            jax.ShapeDtypeStruct((N, H1), jnp.float32),
        ],
    )(x, W1, Asd1)


_CA = 2000
_EPW = E // NW


def _sca1_body(src_hbm, dst_hbm, saA_hbm, saB_hbm, evT_hbm, den_hbm,
               idxS, idxD, rowsA, rowsB, ev2d, evT2d, sem, den_sp):
    c = lax.axis_index("c")
    s = lax.axis_index("s")
    wid = c * NS + s

    iota = lax.iota(jnp.int32, 16)
    ioh = iota // 8
    iol = iota % 8
    zeros16 = jnp.zeros((16,), jnp.float32)

    def _z(j, _):
        plsc.store_scatter(ev2d, [2 * j + ioh, iol], zeros16)
        return 0
    lax.fori_loop(0, _CA // 2, _z, 0, unroll=8)

    @pl.when(s < 5)
    def _():
        pltpu.sync_copy(ev2d, den_sp.at[pl.ds(s * _CA, _CA)])
    plsc.subcore_barrier()

    def chunk_body(k, _):
        base = wid * _EPW + k * _CA
        pltpu.sync_copy(src_hbm.at[pl.ds(base, _CA)], idxS)
        pltpu.sync_copy(dst_hbm.at[pl.ds(base, _CA)], idxD)
        cpA = pltpu.async_copy(saA_hbm.at[idxS], rowsA, sem)
        cpA.wait()
        cpB = pltpu.async_copy(saB_hbm.at[idxD], rowsB, sem)
        cpB.wait()

        def ebody(j, _):
            va = plsc.load_gather(rowsA, [2 * j + ioh, iol])
            vb = plsc.load_gather(rowsB, [2 * j + ioh, iol])
            lg = va + vb
            lg = jnp.maximum(lg, 0.2 * lg)
            e = jnp.exp(lg)
            plsc.store_scatter(ev2d, [2 * j + ioh, iol], e)
            return 0
        lax.fori_loop(0, _CA // 2, ebody, 0, unroll=8)

        def tbody(j, _):
            row = 16 * j + iota
            for hd in range(H1):
                v = plsc.load_gather(ev2d, [row, jnp.full((16,), hd, jnp.int32)])
                evT2d[hd, pl.ds(16 * j, 16)] = v
            return 0
        lax.fori_loop(0, _CA // 16, tbody, 0, unroll=2)
        for hd in range(H1):
            pltpu.sync_copy(evT2d.at[hd], evT_hbm.at[hd, pl.ds(base, _CA)])

        pltpu.sync_copy(ev2d, den_sp.at[idxD], add=True)
        return 0

    lax.fori_loop(0, _EPW // _CA, chunk_body, 0)

    plsc.subcore_barrier()
    @pl.when(s < 5)
    def _():
        pltpu.sync_copy(den_sp.at[pl.ds(s * _CA, _CA)],
                        den_hbm.at[c, pl.ds(s * _CA, _CA)])


def _sca1(src, dst, saA, saB):
    mesh = plsc.VectorSubcoreMesh(core_axis_name="c", subcore_axis_name="s", num_cores=NC, num_subcores=NS)
    f = pl.kernel(
        _sca1_body,
        out_type=[
            jax.ShapeDtypeStruct((H1, E), jnp.float32),
            jax.ShapeDtypeStruct((NC, N, H1), jnp.float32),
        ],
        mesh=mesh,
        compiler_params=pltpu.CompilerParams(use_tc_tiling_on_sc=False, needs_layout_passes=False),
        scratch_types=[
            pltpu.VMEM((_CA,), jnp.int32),
            pltpu.VMEM((_CA,), jnp.int32),
            pltpu.VMEM((_CA, H1), jnp.float32),
            pltpu.VMEM((_CA, H1), jnp.float32),
            pltpu.VMEM((_CA, H1), jnp.float32),
            pltpu.VMEM((H1, _CA), jnp.float32),
            pltpu.SemaphoreType.DMA,
            pltpu.VMEM_SHARED((N, H1), jnp.float32),
        ],
    )
    return f(src, dst, saA, saB)


_CB = 400
_EPS = E // NS
_ZROWS = N // NS


def _scb1_body(src_hbm, dst_hbm, h1f_hbm, evT_hbm, out_hbm,
               idxS, idxD, idxA, rows, ev, zbuf, sem, acc_sp):
    c = lax.axis_index("c")
    s = lax.axis_index("s")

    def _z(j, _):
        for q in range(C1 // 16):
            zbuf[j, pl.ds(16 * q, 16)] = jnp.zeros((16,), jnp.float32)
        return 0
    lax.fori_loop(0, _ZROWS, _z, 0, unroll=8)

    for p in range(H1 // NC):
        hd = c * (H1 // NC) + p
        pltpu.sync_copy(zbuf, acc_sp.at[pl.ds(s * _ZROWS, _ZROWS)])
        plsc.subcore_barrier()

        def chunk_body(k, _):
            base = s * _EPS + k * _CB
            pltpu.sync_copy(src_hbm.at[pl.ds(base, _CB)], idxS)
            pltpu.sync_copy(dst_hbm.at[pl.ds(base, _CB)], idxD)
            pltpu.sync_copy(evT_hbm.at[hd, pl.ds(base, _CB)], ev)

            def abody(j, _):
                idxA[pl.ds(16 * j, 16)] = idxS[pl.ds(16 * j, 16)] + hd * N
                return 0
            lax.fori_loop(0, _CB // 16, abody, 0, unroll=8)

            cp = pltpu.async_copy(h1f_hbm.at[idxA], rows, sem)
            cp.wait()

            def sbody(b, _):
                w = plsc.load_gather(ev, [jnp.full((16,), b, jnp.int32)])
                for q in range(C1 // 16):
                    rows[b, pl.ds(16 * q, 16)] = rows[b, pl.ds(16 * q, 16)] * w
                return 0
            lax.fori_loop(0, _CB, sbody, 0, unroll=4)

            pltpu.sync_copy(rows, acc_sp.at[idxD], add=True)
            return 0

        lax.fori_loop(0, _EPS // _CB, chunk_body, 0)

        plsc.subcore_barrier()
        pltpu.sync_copy(acc_sp.at[pl.ds(s * _ZROWS, _ZROWS)],
                        out_hbm.at[hd, pl.ds(s * _ZROWS, _ZROWS)])
        plsc.subcore_barrier()


def _scb1(src, dst, h1f, evT):
    mesh = plsc.VectorSubcoreMesh(core_axis_name="c", subcore_axis_name="s", num_cores=NC, num_subcores=NS)
    f = pl.kernel(
        _scb1_body,
        out_type=[jax.ShapeDtypeStruct((H1, N, C1), jnp.float32)],
        mesh=mesh,
        compiler_params=pltpu.CompilerParams(use_tc_tiling_on_sc=False, needs_layout_passes=False),
        scratch_types=[
            pltpu.VMEM((_CB,), jnp.int32),
            pltpu.VMEM((_CB,), jnp.int32),
            pltpu.VMEM((_CB,), jnp.int32),
            pltpu.VMEM((_CB, C1), jnp.float32),
            pltpu.VMEM((_CB,), jnp.float32),
            pltpu.VMEM((_ZROWS, C1), jnp.float32),
            pltpu.SemaphoreType.DMA,
            pltpu.VMEM_SHARED((N, C1), jnp.float32),
        ],
    )
    return f(src, dst, h1f, evT)[0]


_BN2 = 1000


def _tc2_body(out1t_ref, denp_ref, h1t_ref, saA_ref, saB_ref, b1_ref,
              w2_ref, asd2_ref, h2_ref, sa2_ref):
    sa_sum = saA_ref[...] + saB_ref[...]
    eself = jnp.exp(jnp.maximum(sa_sum, 0.2 * sa_sum))
    den = denp_ref[0] + denp_ref[1] + eself + 1e-16
    parts = []
    for hd in range(H1):
        o = out1t_ref[hd] + eself[:, hd:hd + 1] * h1t_ref[hd]
        o = o / den[:, hd:hd + 1] + b1_ref[0, hd * C1:(hd + 1) * C1]
        parts.append(jnp.maximum(o, 0.0))
    act = jnp.concatenate(parts, axis=1)
    h2 = jnp.dot(act, w2_ref[...], preferred_element_type=jnp.float32,
                 precision=lax.Precision.HIGHEST)
    h2_ref[...] = h2
    sa2_ref[...] = jnp.dot(h2, asd2_ref[...], preferred_element_type=jnp.float32,
                 precision=lax.Precision.HIGHEST)


def _tc2(out1t, den_p, h1t, saA, saB, b1, W2, Asd2):
    grid = (N // _BN2,)
    return pl.pallas_call(
        _tc2_body,
        grid=grid,
        in_specs=[
            pl.BlockSpec((H1, _BN2, C1), lambda i: (0, i, 0)),
            pl.BlockSpec((NC, _BN2, H1), lambda i: (0, i, 0)),
            pl.BlockSpec((H1, _BN2, C1), lambda i: (0, i, 0)),
            pl.BlockSpec((_BN2, H1), lambda i: (i, 0)),
            pl.BlockSpec((_BN2, H1), lambda i: (i, 0)),
            pl.BlockSpec((1, D1), lambda i: (0, 0)),
            pl.BlockSpec((D1, C2), lambda i: (0, 0)),
            pl.BlockSpec((C2, 8), lambda i: (0, 0)),
        ],
        out_specs=[
            pl.BlockSpec((_BN2, C2), lambda i: (i, 0)),
            pl.BlockSpec((_BN2, 8), lambda i: (i, 0)),
        ],
        out_shape=[
            jax.ShapeDtypeStruct((N, C2), jnp.float32),
            jax.ShapeDtypeStruct((N, 8), jnp.float32),
        ],
    )(out1t, den_p, h1t, saA, saB, b1, W2, Asd2)


_C2K = 2000
_ZR2 = 640


def _scab2_body(src_hbm, dst_hbm, sa2_hbm, h2_hbm, out_hbm, den_hbm,
                idxS, idxD, rowsS, rowsD, rows, ev, sem, acc_sp, den_sp):
    c = lax.axis_index("c")
    s = lax.axis_index("s")
    wid = c * NS + s
    iota = lax.iota(jnp.int32, 16)

    if True:
        def _z(j, _):
            for q in range(C2 // 16):
                rows[j, pl.ds(16 * q, 16)] = jnp.zeros((16,), jnp.float32)
            return 0
        lax.fori_loop(0, _ZR2, _z, 0, unroll=8)
        def _z2(j, _):
            ev[pl.ds(16 * j, 16)] = jnp.zeros((16,), jnp.float32)
            return 0
        lax.fori_loop(0, _ZR2 // 16, _z2, 0, unroll=8)

        pltpu.sync_copy(rows.at[pl.ds(0, _ZR2)],
                        acc_sp.at[pl.ds(s * _ZR2, _ZR2)])
        pltpu.sync_copy(ev.at[pl.ds(0, _ZR2)], den_sp.at[pl.ds(s * _ZR2, _ZR2)])
        plsc.subcore_barrier()

        def chunk_body(k, _):
            base = wid * _EPW + k * _C2K
            pltpu.sync_copy(src_hbm.at[pl.ds(base, _C2K)], idxS)
            pltpu.sync_copy(dst_hbm.at[pl.ds(base, _C2K)], idxD)
            pltpu.async_copy(sa2_hbm.at[idxS], rowsS, sem).wait()
            pltpu.async_copy(sa2_hbm.at[idxD], rowsD, sem).wait()
            pltpu.async_copy(h2_hbm.at[idxS], rows, sem).wait()

            def ebody(j, _):
                row = 16 * j + iota
                va = plsc.load_gather(rowsS, [row, jnp.zeros((16,), jnp.int32)])
                vb = plsc.load_gather(rowsD, [row, jnp.ones((16,), jnp.int32)])
                lg = va + vb
                lg = jnp.maximum(lg, 0.2 * lg)
                ev[pl.ds(16 * j, 16)] = jnp.exp(lg)
                return 0
            lax.fori_loop(0, _C2K // 16, ebody, 0, unroll=4)

            pltpu.sync_copy(ev, den_sp.at[idxD], add=True)

            def sbody(b, _):
                w = plsc.load_gather(ev, [jnp.full((16,), b, jnp.int32)])
                rows[b, pl.ds(0, 16)] = rows[b, pl.ds(0, 16)] * w
                rows[b, pl.ds(16, 16)] = rows[b, pl.ds(16, 16)] * w
                return 0
            lax.fori_loop(0, _C2K, sbody, 0, unroll=4)

            pltpu.sync_copy(rows, acc_sp.at[idxD], add=True)
            return 0

        lax.fori_loop(0, _EPW // _C2K, chunk_body, 0)

        plsc.subcore_barrier()
        base = s * _ZR2
        @pl.when(s < NS - 1)
        def _():
            pltpu.sync_copy(acc_sp.at[pl.ds(base, _ZR2)],
                            out_hbm.at[c, pl.ds(base, _ZR2)])
            pltpu.sync_copy(den_sp.at[pl.ds(base, _ZR2)],
                            den_hbm.at[c, pl.ds(base, _ZR2)])
        @pl.when(s == NS - 1)
        def _():
            last = N - _ZR2 * (NS - 1)
            pltpu.sync_copy(acc_sp.at[pl.ds(base, last)],
                            out_hbm.at[c, pl.ds(base, last)])
            pltpu.sync_copy(den_sp.at[pl.ds(base, last)],
                            den_hbm.at[c, pl.ds(base, last)])


def _scab2(src, dst, sa2, h2):
    mesh = plsc.VectorSubcoreMesh(core_axis_name="c", subcore_axis_name="s", num_cores=NC, num_subcores=NS)
    f = pl.kernel(
        _scab2_body,
        out_type=[
            jax.ShapeDtypeStruct((NC, N, C2), jnp.float32),
            jax.ShapeDtypeStruct((NC, N), jnp.float32),
        ],
        mesh=mesh,
        compiler_params=pltpu.CompilerParams(use_tc_tiling_on_sc=False, needs_layout_passes=False),
        scratch_types=[
            pltpu.VMEM((_C2K,), jnp.int32),
            pltpu.VMEM((_C2K,), jnp.int32),
            pltpu.VMEM((_C2K, 8), jnp.float32),
            pltpu.VMEM((_C2K, 8), jnp.float32),
            pltpu.VMEM((_C2K, C2), jnp.float32),
            pltpu.VMEM((_C2K,), jnp.float32),
            pltpu.SemaphoreType.DMA,
            pltpu.VMEM_SHARED((N, C2), jnp.float32),
            pltpu.VMEM_SHARED((N,), jnp.float32),
        ],
    )
    return f(src, dst, sa2, h2)


_BN3 = 1000


def _tc3_body(out2p_ref, den2p_ref, h2_ref, sa2_ref, b2_ref, fcw_ref, fcb_ref,
              emb_ref, out_ref):
    sa_sum = sa2_ref[:, 0:1] + sa2_ref[:, 1:2]
    eself = jnp.exp(jnp.maximum(sa_sum, 0.2 * sa_sum))
    den = den2p_ref[:, 0:1] + den2p_ref[:, 1:2] + eself + 1e-16
    o = out2p_ref[0] + out2p_ref[1] + eself * h2_ref[...]
    emb = jnp.maximum(o / den + b2_ref[0, :], 0.0)
    emb_ref[...] = emb
    out_ref[...] = jnp.dot(emb, fcw_ref[...],
                           preferred_element_type=jnp.float32,
                 precision=lax.Precision.HIGHEST) + fcb_ref[0, :]


def _tc3(out2p, den2p, h2, sa2, b2, fc_W, fc_b):
    grid = (N // _BN3,)
    return pl.pallas_call(
        _tc3_body,
        grid=grid,
        in_specs=[
            pl.BlockSpec((NC, _BN3, C2), lambda i: (0, i, 0)),
            pl.BlockSpec((_BN3, NC), lambda i: (i, 0)),
            pl.BlockSpec((_BN3, C2), lambda i: (i, 0)),
            pl.BlockSpec((_BN3, 8), lambda i: (i, 0)),
            pl.BlockSpec((1, C2), lambda i: (0, 0)),
            pl.BlockSpec((C2, DOUT), lambda i: (0, 0)),
            pl.BlockSpec((1, DOUT), lambda i: (0, 0)),
        ],
        out_specs=[
            pl.BlockSpec((_BN3, C2), lambda i: (i, 0)),
            pl.BlockSpec((_BN3, DOUT), lambda i: (i, 0)),
        ],
        out_shape=[
            jax.ShapeDtypeStruct((N, C2), jnp.float32),
            jax.ShapeDtypeStruct((N, DOUT), jnp.float32),
        ],
    )(out2p, den2p, h2, sa2, b2, fc_W, fc_b)


def kernel(x, edge_index, W1, a_src1, a_dst1, b1, W2, a_src2, a_dst2, b2,
           fc_W, fc_b):
    src = edge_index[0]
    dst = edge_index[1]

    eye = jnp.eye(H1, dtype=jnp.float32)
    AsdA = (a_src1[:, :, None] * eye[:, None, :]).reshape(D1, H1)
    AsdB = (a_dst1[:, :, None] * eye[:, None, :]).reshape(D1, H1)
    Asd1 = jnp.concatenate([AsdA, AsdB], axis=1)
    Asd2 = jnp.concatenate(
        [a_src2[0][:, None], a_dst2[0][:, None],
         jnp.zeros((C2, 6), jnp.float32)], axis=1)

    h1t, saA, saB = _tc1(x, W1, Asd1)

    evT, den_p = _sca1(src, dst, saA, saB)
    h1f = h1t.reshape(H1 * N, C1)
    out1t = _scb1(src, dst, h1f, evT)
    h2, sa2 = _tc2(out1t, den_p, h1t, saA, saB, b1[None, :], W2, Asd2)

    out2p, den2p = _scab2(src, dst, sa2, h2)
    emb, out = _tc3(out2p, den2p.T, h2, sa2, b2[None, :], fc_W,
                    fc_b[None, :])
    return (emb, out)


# speedup vs baseline: 31.4841x; 1.1422x over previous
import functools

import jax
import jax.numpy as jnp
from jax import lax
from jax.experimental import pallas as pl
from jax.experimental.pallas import tpu as pltpu
from jax.experimental.pallas import tpu_sc as plsc

N = 10000
E = 320000
DIN = 128
H1 = 8
C1 = 64
D1 = H1 * C1
C2 = 32
DOUT = 64

NC = 2
NS = 16
NW = NC * NS


_BN1 = 1000


def _tc1_body(x_ref, w1_ref, asd_ref, h1t_ref, saA_ref, saB_ref):
    h = jnp.dot(x_ref[...], w1_ref[...], preferred_element_type=jnp.float32,
                 precision=lax.Precision.HIGHEST)
    sa = jnp.dot(h, asd_ref[...], preferred_element_type=jnp.float32,
                 precision=lax.Precision.HIGHEST)
    saA_ref[...] = sa[:, :H1]
    saB_ref[...] = sa[:, H1:]
    for p in range(H1 // 2):
        h1t_ref[p] = h[:, p * 2 * C1:(p + 1) * 2 * C1]


def _tc1(x, W1, Asd1):
    grid = (N // _BN1,)
    return pl.pallas_call(
        _tc1_body,
        grid=grid,
        in_specs=[
            pl.BlockSpec((_BN1, DIN), lambda i: (i, 0)),
            pl.BlockSpec((DIN, D1), lambda i: (0, 0)),
            pl.BlockSpec((D1, 2 * H1), lambda i: (0, 0)),
        ],
        out_specs=[
            pl.BlockSpec((H1 // 2, _BN1, 2 * C1), lambda i: (0, i, 0)),
            pl.BlockSpec((_BN1, H1), lambda i: (i, 0)),
            pl.BlockSpec((_BN1, H1), lambda i: (i, 0)),
        ],
        out_shape=[
            jax.ShapeDtypeStruct((H1 // 2, N, 2 * C1), jnp.float32),
            jax.ShapeDtypeStruct((N, H1), jnp.float32),
            jax.ShapeDtypeStruct((N, H1), jnp.float32),
        ],
    )(x, W1, Asd1)


_CA = 2000
_EPW = E // NW


def _sca1_body(src_hbm, dst_hbm, saA_hbm, saB_hbm, evT_hbm, den_hbm,
               idxS, idxD, rowsA, rowsB, ev2d, evT2d, sem, den_sp):
    c = lax.axis_index("c")
    s = lax.axis_index("s")
    wid = c * NS + s

    iota = lax.iota(jnp.int32, 16)
    ioh = iota // 8
    iol = iota % 8
    zeros16 = jnp.zeros((16,), jnp.float32)

    def _z(j, _):
        plsc.store_scatter(ev2d, [2 * j + ioh, iol], zeros16)
        return 0
    lax.fori_loop(0, _CA // 2, _z, 0, unroll=8)

    @pl.when(s < 5)
    def _():
        pltpu.sync_copy(ev2d, den_sp.at[pl.ds(s * _CA, _CA)])
    plsc.subcore_barrier()

    def chunk_body(k, _):
        base = wid * _EPW + k * _CA
        pltpu.sync_copy(src_hbm.at[pl.ds(base, _CA)], idxS)
        pltpu.sync_copy(dst_hbm.at[pl.ds(base, _CA)], idxD)
        cpA = pltpu.async_copy(saA_hbm.at[idxS], rowsA, sem)
        cpA.wait()
        cpB = pltpu.async_copy(saB_hbm.at[idxD], rowsB, sem)
        cpB.wait()

        def ebody(j, _):
            va = plsc.load_gather(rowsA, [2 * j + ioh, iol])
            vb = plsc.load_gather(rowsB, [2 * j + ioh, iol])
            lg = va + vb
            lg = jnp.maximum(lg, 0.2 * lg)
            e = jnp.exp(lg)
            plsc.store_scatter(ev2d, [2 * j + ioh, iol], e)
            return 0
        lax.fori_loop(0, _CA // 2, ebody, 0, unroll=8)

        def tbody(j, _):
            row = 16 * j + iota
            for hd in range(H1):
                v = plsc.load_gather(ev2d, [row, jnp.full((16,), hd, jnp.int32)])
                evT2d[hd, pl.ds(16 * j, 16)] = v
            return 0
        lax.fori_loop(0, _CA // 16, tbody, 0, unroll=2)
        for hd in range(H1):
            pltpu.sync_copy(evT2d.at[hd], evT_hbm.at[hd, pl.ds(base, _CA)])

        pltpu.sync_copy(ev2d, den_sp.at[idxD], add=True)
        return 0

    lax.fori_loop(0, _EPW // _CA, chunk_body, 0)

    plsc.subcore_barrier()
    @pl.when(s < 5)
    def _():
        pltpu.sync_copy(den_sp.at[pl.ds(s * _CA, _CA)],
                        den_hbm.at[c, pl.ds(s * _CA, _CA)])


def _sca1(src, dst, saA, saB):
    mesh = plsc.VectorSubcoreMesh(core_axis_name="c", subcore_axis_name="s", num_cores=NC, num_subcores=NS)
    f = pl.kernel(
        _sca1_body,
        out_type=[
            jax.ShapeDtypeStruct((H1, E), jnp.float32),
            jax.ShapeDtypeStruct((NC, N, H1), jnp.float32),
        ],
        mesh=mesh,
        compiler_params=pltpu.CompilerParams(use_tc_tiling_on_sc=False, needs_layout_passes=False),
        scratch_types=[
            pltpu.VMEM((_CA,), jnp.int32),
            pltpu.VMEM((_CA,), jnp.int32),
            pltpu.VMEM((_CA, H1), jnp.float32),
            pltpu.VMEM((_CA, H1), jnp.float32),
            pltpu.VMEM((_CA, H1), jnp.float32),
            pltpu.VMEM((H1, _CA), jnp.float32),
            pltpu.SemaphoreType.DMA,
            pltpu.VMEM_SHARED((N, H1), jnp.float32),
        ],
    )
    return f(src, dst, saA, saB)


_CB = 800
_EPS = E // NS
_ZROWS = N // NS
_CP = 2 * C1


def _scb1_body(src_hbm, dst_hbm, h1f_hbm, evT_hbm, out_hbm,
               idxS, idxD, idxA, rows, ev, sem, semi, acc_sp):
    c = lax.axis_index("c")
    s = lax.axis_index("s")

    for p in range(H1 // NC):
        hd = c * (H1 // NC) + p
        pair = hd // 2

        def _z(j, _):
            for q in range(C1 // 16):
                rows[j, pl.ds(16 * q, 16)] = jnp.zeros((16,), jnp.float32)
            return 0
        lax.fori_loop(0, _ZROWS, _z, 0, unroll=8)
        pltpu.sync_copy(rows.at[pl.ds(0, _ZROWS)],
                        acc_sp.at[pl.ds(s * _ZROWS, _ZROWS)])
        plsc.subcore_barrier()

        def chunk_body(k, _):
            base = s * _EPS + k * _CB
            c1 = pltpu.async_copy(src_hbm.at[pl.ds(base, _CB)], idxS, semi)
            c2 = pltpu.async_copy(dst_hbm.at[pl.ds(base, _CB)], idxD, semi)
            c3 = pltpu.async_copy(evT_hbm.at[hd, pl.ds(base, _CB)], ev, semi)
            c1.wait(); c2.wait(); c3.wait()

            def abody(j, _):
                idxA[pl.ds(16 * j, 16)] = (
                    idxS[pl.ds(16 * j, 16)] * 2 + (2 * pair * N + hd % 2))
                return 0
            lax.fori_loop(0, _CB // 16, abody, 0, unroll=8)

            cp = pltpu.async_copy(h1f_hbm.at[idxA], rows, sem)
            cp.wait()

            def sbody(b, _):
                w = plsc.load_gather(ev, [jnp.full((16,), b, jnp.int32)])
                for q in range(C1 // 16):
                    rows[b, pl.ds(16 * q, 16)] = rows[b, pl.ds(16 * q, 16)] * w
                return 0
            lax.fori_loop(0, _CB, sbody, 0, unroll=4)

            pltpu.sync_copy(rows, acc_sp.at[idxD], add=True)
            return 0

        lax.fori_loop(0, _EPS // _CB, chunk_body, 0)

        plsc.subcore_barrier()
        pltpu.sync_copy(acc_sp.at[pl.ds(s * _ZROWS, _ZROWS)],
                        out_hbm.at[hd, pl.ds(s * _ZROWS, _ZROWS)])
        plsc.subcore_barrier()


def _scb1(src, dst, h1f, evT):
    mesh = plsc.VectorSubcoreMesh(core_axis_name="c", subcore_axis_name="s", num_cores=NC, num_subcores=NS)
    f = pl.kernel(
        _scb1_body,
        out_type=[jax.ShapeDtypeStruct((H1, N, C1), jnp.float32)],
        mesh=mesh,
        compiler_params=pltpu.CompilerParams(use_tc_tiling_on_sc=False, needs_layout_passes=False),
        scratch_types=[
            pltpu.VMEM((_CB,), jnp.int32),
            pltpu.VMEM((_CB,), jnp.int32),
            pltpu.VMEM((_CB,), jnp.int32),
            pltpu.VMEM((_CB, C1), jnp.float32),
            pltpu.VMEM((_CB,), jnp.float32),
            pltpu.SemaphoreType.DMA,
            pltpu.SemaphoreType.DMA,
            pltpu.VMEM_SHARED((N, C1), jnp.float32),
        ],
    )
    return f(src, dst, h1f, evT)[0]


_BN2 = 1000


def _tc2_body(out1t_ref, denp_ref, h1t_ref, saA_ref, saB_ref, b1_ref,
              w2_ref, asd2_ref, h2_ref, sa2_ref):
    sa_sum = saA_ref[...] + saB_ref[...]
    eself = jnp.exp(jnp.maximum(sa_sum, 0.2 * sa_sum))
    den = denp_ref[0] + denp_ref[1] + eself + 1e-16
    parts = []
    for hd in range(H1):
        p, q = hd // 2, (hd % 2) * C1
        o = (out1t_ref[hd]
             + eself[:, hd:hd + 1] * h1t_ref[p][:, q:q + C1])
        o = o / den[:, hd:hd + 1] + b1_ref[0, hd * C1:(hd + 1) * C1]
        parts.append(jnp.maximum(o, 0.0))
    act = jnp.concatenate(parts, axis=1)
    h2 = jnp.dot(act, w2_ref[...], preferred_element_type=jnp.float32,
                 precision=lax.Precision.HIGHEST)
    h2_ref[...] = h2
    sa2_ref[...] = jnp.dot(h2, asd2_ref[...], preferred_element_type=jnp.float32,
                 precision=lax.Precision.HIGHEST)


def _tc2(out1t, den_p, h1t, saA, saB, b1, W2, Asd2):
    grid = (N // _BN2,)
    return pl.pallas_call(
        _tc2_body,
        grid=grid,
        in_specs=[
            pl.BlockSpec((H1, _BN2, C1), lambda i: (0, i, 0)),
            pl.BlockSpec((NC, _BN2, H1), lambda i: (0, i, 0)),
            pl.BlockSpec((H1 // 2, _BN2, 2 * C1), lambda i: (0, i, 0)),
            pl.BlockSpec((_BN2, H1), lambda i: (i, 0)),
            pl.BlockSpec((_BN2, H1), lambda i: (i, 0)),
            pl.BlockSpec((1, D1), lambda i: (0, 0)),
            pl.BlockSpec((D1, C2), lambda i: (0, 0)),
            pl.BlockSpec((C2, 8), lambda i: (0, 0)),
        ],
        out_specs=[
            pl.BlockSpec((_BN2, C2), lambda i: (i, 0)),
            pl.BlockSpec((_BN2, 8), lambda i: (i, 0)),
        ],
        out_shape=[
            jax.ShapeDtypeStruct((N, C2), jnp.float32),
            jax.ShapeDtypeStruct((N, 8), jnp.float32),
        ],
    )(out1t, den_p, h1t, saA, saB, b1, W2, Asd2)


_C2K = 400
_ZR2 = 640


def _scab2_body(src_hbm, dst_hbm, sa2_hbm, h2_hbm, out_hbm, den_hbm,
                idxS, idxD, rowsS, rowsD, rows, ev, sem, acc_sp, den_sp):
    c = lax.axis_index("c")
    s = lax.axis_index("s")
    wid = c * NS + s
    iota = lax.iota(jnp.int32, 16)

    if True:
        def _z(j, _):
            for q in range(C2 // 16):
                rows[j, pl.ds(16 * q, 16)] = jnp.zeros((16,), jnp.float32)
            return 0
        lax.fori_loop(0, _ZR2, _z, 0, unroll=8)
        def _z2(j, _):
            ev[pl.ds(16 * j, 16)] = jnp.zeros((16,), jnp.float32)
            return 0
        lax.fori_loop(0, _ZR2 // 16, _z2, 0, unroll=8)

        pltpu.sync_copy(rows.at[pl.ds(0, _ZR2)],
                        acc_sp.at[pl.ds(s * _ZR2, _ZR2)])
        pltpu.sync_copy(ev.at[pl.ds(0, _ZR2)], den_sp.at[pl.ds(s * _ZR2, _ZR2)])
        plsc.subcore_barrier()

        def chunk_body(k, _):
            base = wid * _EPW + k * _C2K
            pltpu.sync_copy(src_hbm.at[pl.ds(base, _C2K)], idxS)
            pltpu.sync_copy(dst_hbm.at[pl.ds(base, _C2K)], idxD)
            pltpu.async_copy(sa2_hbm.at[idxS], rowsS, sem).wait()
            pltpu.async_copy(sa2_hbm.at[idxD], rowsD, sem).wait()
            pltpu.async_copy(h2_hbm.at[idxS], rows, sem).wait()

            def ebody(j, _):
                row = 16 * j + iota
                va = plsc.load_gather(rowsS, [row, jnp.zeros((16,), jnp.int32)])
                vb = plsc.load_gather(rowsD, [row, jnp.ones((16,), jnp.int32)])
                lg = va + vb
                lg = jnp.maximum(lg, 0.2 * lg)
                ev[pl.ds(16 * j, 16)] = jnp.exp(lg)
                return 0
            lax.fori_loop(0, _C2K // 16, ebody, 0, unroll=4)

            pltpu.sync_copy(ev, den_sp.at[idxD], add=True)

            def sbody(b, _):
                w = plsc.load_gather(ev, [jnp.full((16,), b, jnp.int32)])
                rows[b, pl.ds(0, 16)] = rows[b, pl.ds(0, 16)] * w
                rows[b, pl.ds(16, 16)] = rows[b, pl.ds(16, 16)] * w
                return 0
            lax.fori_loop(0, _C2K, sbody, 0, unroll=4)

            pltpu.sync_copy(rows, acc_sp.at[idxD], add=True)
            return 0

        lax.fori_loop(0, _EPW // _C2K, chunk_body, 0)

        plsc.subcore_barrier()
        base = s * _ZR2
        @pl.when(s < NS - 1)
        def _():
            pltpu.sync_copy(acc_sp.at[pl.ds(base, _ZR2)],
                            out_hbm.at[c, pl.ds(base, _ZR2)])
            pltpu.sync_copy(den_sp.at[pl.ds(base, _ZR2)],
                            den_hbm.at[c, pl.ds(base, _ZR2)])
        @pl.when(s == NS - 1)
        def _():
            last = N - _ZR2 * (NS - 1)
            pltpu.sync_copy(acc_sp.at[pl.ds(base, last)],
                            out_hbm.at[c, pl.ds(base, last)])
            pltpu.sync_copy(den_sp.at[pl.ds(base, last)],
                            den_hbm.at[c, pl.ds(base, last)])


def _scab2(src, dst, sa2, h2):
    mesh = plsc.VectorSubcoreMesh(core_axis_name="c", subcore_axis_name="s", num_cores=NC, num_subcores=NS)
    f = pl.kernel(
        _scab2_body,
        out_type=[
            jax.ShapeDtypeStruct((NC, N, C2), jnp.float32),
            jax.ShapeDtypeStruct((NC, N), jnp.float32),
        ],
        mesh=mesh,
        compiler_params=pltpu.CompilerParams(use_tc_tiling_on_sc=False, needs_layout_passes=False),
        scratch_types=[
            pltpu.VMEM((_C2K,), jnp.int32),
            pltpu.VMEM((_C2K,), jnp.int32),
            pltpu.VMEM((_C2K, 8), jnp.float32),
            pltpu.VMEM((_C2K, 8), jnp.float32),
            pltpu.VMEM((_C2K, C2), jnp.float32),
            pltpu.VMEM((_C2K,), jnp.float32),
            pltpu.SemaphoreType.DMA,
            pltpu.VMEM_SHARED((N, C2), jnp.float32),
            pltpu.VMEM_SHARED((N,), jnp.float32),
        ],
    )
    return f(src, dst, sa2, h2)


_BN3 = 1000


def _tc3_body(out2p_ref, den2p_ref, h2_ref, sa2_ref, b2_ref, fcw_ref, fcb_ref,
              emb_ref, out_ref):
    sa_sum = sa2_ref[:, 0:1] + sa2_ref[:, 1:2]
    eself = jnp.exp(jnp.maximum(sa_sum, 0.2 * sa_sum))
    den = den2p_ref[:, 0:1] + den2p_ref[:, 1:2] + eself + 1e-16
    o = out2p_ref[0] + out2p_ref[1] + eself * h2_ref[...]
    emb = jnp.maximum(o / den + b2_ref[0, :], 0.0)
    emb_ref[...] = emb
    out_ref[...] = jnp.dot(emb, fcw_ref[...],
                           preferred_element_type=jnp.float32,
                 precision=lax.Precision.HIGHEST) + fcb_ref[0, :]


def _tc3(out2p, den2p, h2, sa2, b2, fc_W, fc_b):
    grid = (N // _BN3,)
    return pl.pallas_call(
        _tc3_body,
        grid=grid,
        in_specs=[
            pl.BlockSpec((NC, _BN3, C2), lambda i: (0, i, 0)),
            pl.BlockSpec((_BN3, NC), lambda i: (i, 0)),
            pl.BlockSpec((_BN3, C2), lambda i: (i, 0)),
            pl.BlockSpec((_BN3, 8), lambda i: (i, 0)),
            pl.BlockSpec((1, C2), lambda i: (0, 0)),
            pl.BlockSpec((C2, DOUT), lambda i: (0, 0)),
            pl.BlockSpec((1, DOUT), lambda i: (0, 0)),
        ],
        out_specs=[
            pl.BlockSpec((_BN3, C2), lambda i: (i, 0)),
            pl.BlockSpec((_BN3, DOUT), lambda i: (i, 0)),
        ],
        out_shape=[
            jax.ShapeDtypeStruct((N, C2), jnp.float32),
            jax.ShapeDtypeStruct((N, DOUT), jnp.float32),
        ],
    )(out2p, den2p, h2, sa2, b2, fc_W, fc_b)


def kernel(x, edge_index, W1, a_src1, a_dst1, b1, W2, a_src2, a_dst2, b2,
           fc_W, fc_b):
    src = edge_index[0]
    dst = edge_index[1]

    eye = jnp.eye(H1, dtype=jnp.float32)
    AsdA = (a_src1[:, :, None] * eye[:, None, :]).reshape(D1, H1)
    AsdB = (a_dst1[:, :, None] * eye[:, None, :]).reshape(D1, H1)
    Asd1 = jnp.concatenate([AsdA, AsdB], axis=1)
    Asd2 = jnp.concatenate(
        [a_src2[0][:, None], a_dst2[0][:, None],
         jnp.zeros((C2, 6), jnp.float32)], axis=1)

    h1t, saA, saB = _tc1(x, W1, Asd1)

    evT, den_p = _sca1(src, dst, saA, saB)
    h1f = h1t.reshape(H1 * N, C1)
    out1t = _scb1(src, dst, h1f, evT)
    h2, sa2 = _tc2(out1t, den_p, h1t, saA, saB, b1[None, :], W2, Asd2)

    out2p, den2p = _scab2(src, dst, sa2, h2)
    emb, out = _tc3(out2p, den2p.T, h2, sa2, b2[None, :], fc_W,
                    fc_b[None, :])
    return (emb, out)


# speedup vs baseline: 35.5732x; 1.1299x over previous
import functools

import jax
import jax.numpy as jnp
from jax import lax
from jax.experimental import pallas as pl
from jax.experimental.pallas import tpu as pltpu
from jax.experimental.pallas import tpu_sc as plsc

N = 10000
E = 320000
DIN = 128
H1 = 8
C1 = 64
D1 = H1 * C1
C2 = 32
DOUT = 64

NC = 2
NS = 16
NW = NC * NS


_BN1 = 1000


def _tc1_body(x_ref, w1_ref, asd_ref, h1t_ref, saA_ref, saB_ref):
    h = jnp.dot(x_ref[...], w1_ref[...], preferred_element_type=jnp.float32,
                 precision=lax.Precision.HIGHEST)
    sa = jnp.dot(h, asd_ref[...], preferred_element_type=jnp.float32,
                 precision=lax.Precision.HIGHEST)
    saA_ref[...] = sa[:, :H1]
    saB_ref[...] = sa[:, H1:]
    for p in range(H1 // 2):
        h1t_ref[p] = h[:, p * 2 * C1:(p + 1) * 2 * C1]


def _tc1(x, W1, Asd1):
    grid = (N // _BN1,)
    return pl.pallas_call(
        _tc1_body,
        grid=grid,
        in_specs=[
            pl.BlockSpec((_BN1, DIN), lambda i: (i, 0)),
            pl.BlockSpec((DIN, D1), lambda i: (0, 0)),
            pl.BlockSpec((D1, 2 * H1), lambda i: (0, 0)),
        ],
        out_specs=[
            pl.BlockSpec((H1 // 2, _BN1, 2 * C1), lambda i: (0, i, 0)),
            pl.BlockSpec((_BN1, H1), lambda i: (i, 0)),
            pl.BlockSpec((_BN1, H1), lambda i: (i, 0)),
        ],
        out_shape=[
            jax.ShapeDtypeStruct((H1 // 2, N, 2 * C1), jnp.float32),
            jax.ShapeDtypeStruct((N, H1), jnp.float32),
            jax.ShapeDtypeStruct((N, H1), jnp.float32),
        ],
    )(x, W1, Asd1)


_CA = 2000
_EPW = E // NW


def _sca1_body(src_hbm, dst_hbm, saA_hbm, saB_hbm, evT_hbm, den_hbm,
               idxS, idxD, rowsA, rowsB, ev2d, evT2d, sem, den_sp):
    c = lax.axis_index("c")
    s = lax.axis_index("s")
    wid = c * NS + s

    iota = lax.iota(jnp.int32, 16)
    ioh = iota // 8
    iol = iota % 8
    zeros16 = jnp.zeros((16,), jnp.float32)

    def _z(j, _):
        plsc.store_scatter(ev2d, [2 * j + ioh, iol], zeros16)
        return 0
    lax.fori_loop(0, _CA // 2, _z, 0, unroll=8)

    @pl.when(s < 5)
    def _():
        pltpu.sync_copy(ev2d, den_sp.at[pl.ds(s * _CA, _CA)])
    plsc.subcore_barrier()

    def chunk_body(k, _):
        base = wid * _EPW + k * _CA
        pltpu.sync_copy(src_hbm.at[pl.ds(base, _CA)], idxS)
        pltpu.sync_copy(dst_hbm.at[pl.ds(base, _CA)], idxD)
        cpA = pltpu.async_copy(saA_hbm.at[idxS], rowsA, sem)
        cpA.wait()
        cpB = pltpu.async_copy(saB_hbm.at[idxD], rowsB, sem)
        cpB.wait()

        def ebody(j, _):
            va = plsc.load_gather(rowsA, [2 * j + ioh, iol])
            vb = plsc.load_gather(rowsB, [2 * j + ioh, iol])
            lg = va + vb
            lg = jnp.maximum(lg, 0.2 * lg)
            e = jnp.exp(lg)
            plsc.store_scatter(ev2d, [2 * j + ioh, iol], e)
            return 0
        lax.fori_loop(0, _CA // 2, ebody, 0, unroll=8)

        def tbody(j, _):
            row = 16 * j + iota
            for hd in range(H1):
                v = plsc.load_gather(ev2d, [row, jnp.full((16,), hd, jnp.int32)])
                evT2d[hd, pl.ds(16 * j, 16)] = v
            return 0
        lax.fori_loop(0, _CA // 16, tbody, 0, unroll=2)
        for hd in range(H1):
            pltpu.sync_copy(evT2d.at[hd], evT_hbm.at[hd, pl.ds(base, _CA)])

        pltpu.sync_copy(ev2d, den_sp.at[idxD], add=True)
        return 0

    lax.fori_loop(0, _EPW // _CA, chunk_body, 0)

    plsc.subcore_barrier()
    @pl.when(s < 5)
    def _():
        pltpu.sync_copy(den_sp.at[pl.ds(s * _CA, _CA)],
                        den_hbm.at[c, pl.ds(s * _CA, _CA)])


def _sca1(src, dst, saA, saB):
    mesh = plsc.VectorSubcoreMesh(core_axis_name="c", subcore_axis_name="s", num_cores=NC, num_subcores=NS)
    f = pl.kernel(
        _sca1_body,
        out_type=[
            jax.ShapeDtypeStruct((H1, E), jnp.float32),
            jax.ShapeDtypeStruct((NC, N, H1), jnp.float32),
        ],
        mesh=mesh,
        compiler_params=pltpu.CompilerParams(use_tc_tiling_on_sc=False, needs_layout_passes=False),
        scratch_types=[
            pltpu.VMEM((_CA,), jnp.int32),
            pltpu.VMEM((_CA,), jnp.int32),
            pltpu.VMEM((_CA, H1), jnp.float32),
            pltpu.VMEM((_CA, H1), jnp.float32),
            pltpu.VMEM((_CA, H1), jnp.float32),
            pltpu.VMEM((H1, _CA), jnp.float32),
            pltpu.SemaphoreType.DMA,
            pltpu.VMEM_SHARED((N, H1), jnp.float32),
        ],
    )
    return f(src, dst, saA, saB)


_CB = 400
_EPS = E // NS
_NCK = _EPS // _CB
_ZROWS = N // NS


def _scb1_body(src_hbm, dst_hbm, h1f_hbm, evT_hbm, out_hbm,
               idxSa, idxDa, idxAa, rowsa, eva,
               idxSb, idxDb, idxAb, rowsb, evb,
               sema, semb, semi, acc_sp):
    c = lax.axis_index("c")
    s = lax.axis_index("s")

    for p in range(H1 // NC):
        hd = c * (H1 // NC) + p
        pair = hd // 2
        rowoff = 2 * pair * N + (hd % 2)

        def issue(k, idxS, idxD, idxA, rows, ev, sem):
            base = s * _EPS + k * _CB
            c1 = pltpu.async_copy(src_hbm.at[pl.ds(base, _CB)], idxS, semi)
            c2 = pltpu.async_copy(dst_hbm.at[pl.ds(base, _CB)], idxD, semi)
            c3 = pltpu.async_copy(evT_hbm.at[hd, pl.ds(base, _CB)], ev, semi)
            c1.wait(); c2.wait(); c3.wait()

            def abody(j, _):
                idxA[pl.ds(16 * j, 16)] = idxS[pl.ds(16 * j, 16)] * 2 + rowoff
                return 0
            lax.fori_loop(0, _CB // 16, abody, 0, unroll=8)
            pltpu.async_copy(h1f_hbm.at[idxA], rows, sem)

        def drain(idxA, rows, sem):
            pltpu.make_async_copy(h1f_hbm.at[idxA], rows, sem).wait()

        def scale_scatter(idxD, rows, ev):
            def sbody(b, _):
                w = plsc.load_gather(ev, [jnp.full((16,), b, jnp.int32)])
                for q in range(C1 // 16):
                    rows[b, pl.ds(16 * q, 16)] = rows[b, pl.ds(16 * q, 16)] * w
                return 0
            lax.fori_loop(0, _CB, sbody, 0, unroll=4)
            pltpu.sync_copy(rows, acc_sp.at[idxD], add=True)

        def _z(j, _):
            for q in range(C1 // 16):
                rowsa[j, pl.ds(16 * q, 16)] = jnp.zeros((16,), jnp.float32)
            return 0
        lax.fori_loop(0, _CB, _z, 0, unroll=8)

        def zcopy(j, _):
            pltpu.sync_copy(rowsa,
                            acc_sp.at[pl.ds(s * _ZROWS + j * _CB, _CB)])
            return 0
        lax.fori_loop(0, _ZROWS // _CB, zcopy, 0)
        pltpu.sync_copy(rowsa.at[pl.ds(0, _ZROWS - _ZROWS // _CB * _CB)],
                        acc_sp.at[pl.ds(s * _ZROWS + _ZROWS // _CB * _CB,
                                        _ZROWS - _ZROWS // _CB * _CB)])
        plsc.subcore_barrier()

        issue(0, idxSa, idxDa, idxAa, rowsa, eva, sema)

        def body2(t, _):
            k0 = 2 * t
            drain(idxAa, rowsa, sema)
            issue(k0 + 1, idxSb, idxDb, idxAb, rowsb, evb, semb)
            scale_scatter(idxDa, rowsa, eva)
            drain(idxAb, rowsb, semb)
            @pl.when(t < _NCK // 2 - 1)
            def _():
                issue(k0 + 2, idxSa, idxDa, idxAa, rowsa, eva, sema)
            scale_scatter(idxDb, rowsb, evb)
            return 0

        lax.fori_loop(0, _NCK // 2, body2, 0)

        plsc.subcore_barrier()
        pltpu.sync_copy(acc_sp.at[pl.ds(s * _ZROWS, _ZROWS)],
                        out_hbm.at[hd, pl.ds(s * _ZROWS, _ZROWS)])
        plsc.subcore_barrier()


def _scb1(src, dst, h1f, evT):
    mesh = plsc.VectorSubcoreMesh(core_axis_name="c", subcore_axis_name="s", num_cores=NC, num_subcores=NS)
    f = pl.kernel(
        _scb1_body,
        out_type=[jax.ShapeDtypeStruct((H1, N, C1), jnp.float32)],
        mesh=mesh,
        compiler_params=pltpu.CompilerParams(use_tc_tiling_on_sc=False, needs_layout_passes=False),
        scratch_types=[
            pltpu.VMEM((_CB,), jnp.int32),
            pltpu.VMEM((_CB,), jnp.int32),
            pltpu.VMEM((_CB,), jnp.int32),
            pltpu.VMEM((_CB, C1), jnp.float32),
            pltpu.VMEM((_CB,), jnp.float32),
            pltpu.VMEM((_CB,), jnp.int32),
            pltpu.VMEM((_CB,), jnp.int32),
            pltpu.VMEM((_CB,), jnp.int32),
            pltpu.VMEM((_CB, C1), jnp.float32),
            pltpu.VMEM((_CB,), jnp.float32),
            pltpu.SemaphoreType.DMA,
            pltpu.SemaphoreType.DMA,
            pltpu.SemaphoreType.DMA,
            pltpu.VMEM_SHARED((N, C1), jnp.float32),
        ],
    )
    return f(src, dst, h1f, evT)[0]


_BN2 = 1000


def _tc2_body(out1t_ref, denp_ref, h1t_ref, saA_ref, saB_ref, b1_ref,
              w2_ref, asd2_ref, h2_ref, sa2_ref):
    sa_sum = saA_ref[...] + saB_ref[...]
    eself = jnp.exp(jnp.maximum(sa_sum, 0.2 * sa_sum))
    den = denp_ref[0] + denp_ref[1] + eself + 1e-16
    parts = []
    for hd in range(H1):
        p, q = hd // 2, (hd % 2) * C1
        o = (out1t_ref[hd]
             + eself[:, hd:hd + 1] * h1t_ref[p][:, q:q + C1])
        o = o / den[:, hd:hd + 1] + b1_ref[0, hd * C1:(hd + 1) * C1]
        parts.append(jnp.maximum(o, 0.0))
    act = jnp.concatenate(parts, axis=1)
    h2 = jnp.dot(act, w2_ref[...], preferred_element_type=jnp.float32,
                 precision=lax.Precision.HIGHEST)
    h2_ref[...] = h2
    sa2_ref[...] = jnp.dot(h2, asd2_ref[...], preferred_element_type=jnp.float32,
                 precision=lax.Precision.HIGHEST)


def _tc2(out1t, den_p, h1t, saA, saB, b1, W2, Asd2):
    grid = (N // _BN2,)
    return pl.pallas_call(
        _tc2_body,
        grid=grid,
        in_specs=[
            pl.BlockSpec((H1, _BN2, C1), lambda i: (0, i, 0)),
            pl.BlockSpec((NC, _BN2, H1), lambda i: (0, i, 0)),
            pl.BlockSpec((H1 // 2, _BN2, 2 * C1), lambda i: (0, i, 0)),
            pl.BlockSpec((_BN2, H1), lambda i: (i, 0)),
            pl.BlockSpec((_BN2, H1), lambda i: (i, 0)),
            pl.BlockSpec((1, D1), lambda i: (0, 0)),
            pl.BlockSpec((D1, C2), lambda i: (0, 0)),
            pl.BlockSpec((C2, 8), lambda i: (0, 0)),
        ],
        out_specs=[
            pl.BlockSpec((_BN2, C2), lambda i: (i, 0)),
            pl.BlockSpec((_BN2, 8), lambda i: (i, 0)),
        ],
        out_shape=[
            jax.ShapeDtypeStruct((N, C2), jnp.float32),
            jax.ShapeDtypeStruct((N, 8), jnp.float32),
        ],
    )(out1t, den_p, h1t, saA, saB, b1, W2, Asd2)


_C2K = 400
_ZR2 = 640


def _scab2_body(src_hbm, dst_hbm, sa2_hbm, h2_hbm, out_hbm, den_hbm,
                idxS, idxD, rowsS, rowsD, rows, ev, sem, acc_sp, den_sp):
    c = lax.axis_index("c")
    s = lax.axis_index("s")
    wid = c * NS + s
    iota = lax.iota(jnp.int32, 16)

    if True:
        def _z(j, _):
            for q in range(C2 // 16):
                rows[j, pl.ds(16 * q, 16)] = jnp.zeros((16,), jnp.float32)
            return 0
        lax.fori_loop(0, _ZR2, _z, 0, unroll=8)
        def _z2(j, _):
            ev[pl.ds(16 * j, 16)] = jnp.zeros((16,), jnp.float32)
            return 0
        lax.fori_loop(0, _ZR2 // 16, _z2, 0, unroll=8)

        pltpu.sync_copy(rows.at[pl.ds(0, _ZR2)],
                        acc_sp.at[pl.ds(s * _ZR2, _ZR2)])
        pltpu.sync_copy(ev.at[pl.ds(0, _ZR2)], den_sp.at[pl.ds(s * _ZR2, _ZR2)])
        plsc.subcore_barrier()

        def chunk_body(k, _):
            base = wid * _EPW + k * _C2K
            pltpu.sync_copy(src_hbm.at[pl.ds(base, _C2K)], idxS)
            pltpu.sync_copy(dst_hbm.at[pl.ds(base, _C2K)], idxD)
            pltpu.async_copy(sa2_hbm.at[idxS], rowsS, sem).wait()
            pltpu.async_copy(sa2_hbm.at[idxD], rowsD, sem).wait()
            pltpu.async_copy(h2_hbm.at[idxS], rows, sem).wait()

            def ebody(j, _):
                row = 16 * j + iota
                va = plsc.load_gather(rowsS, [row, jnp.zeros((16,), jnp.int32)])
                vb = plsc.load_gather(rowsD, [row, jnp.ones((16,), jnp.int32)])
                lg = va + vb
                lg = jnp.maximum(lg, 0.2 * lg)
                ev[pl.ds(16 * j, 16)] = jnp.exp(lg)
                return 0
            lax.fori_loop(0, _C2K // 16, ebody, 0, unroll=4)

            pltpu.sync_copy(ev, den_sp.at[idxD], add=True)

            def sbody(b, _):
                w = plsc.load_gather(ev, [jnp.full((16,), b, jnp.int32)])
                rows[b, pl.ds(0, 16)] = rows[b, pl.ds(0, 16)] * w
                rows[b, pl.ds(16, 16)] = rows[b, pl.ds(16, 16)] * w
                return 0
            lax.fori_loop(0, _C2K, sbody, 0, unroll=4)

            pltpu.sync_copy(rows, acc_sp.at[idxD], add=True)
            return 0

        lax.fori_loop(0, _EPW // _C2K, chunk_body, 0)

        plsc.subcore_barrier()
        base = s * _ZR2
        @pl.when(s < NS - 1)
        def _():
            pltpu.sync_copy(acc_sp.at[pl.ds(base, _ZR2)],
                            out_hbm.at[c, pl.ds(base, _ZR2)])
            pltpu.sync_copy(den_sp.at[pl.ds(base, _ZR2)],
                            den_hbm.at[c, pl.ds(base, _ZR2)])
        @pl.when(s == NS - 1)
        def _():
            last = N - _ZR2 * (NS - 1)
            pltpu.sync_copy(acc_sp.at[pl.ds(base, last)],
                            out_hbm.at[c, pl.ds(base, last)])
            pltpu.sync_copy(den_sp.at[pl.ds(base, last)],
                            den_hbm.at[c, pl.ds(base, last)])


def _scab2(src, dst, sa2, h2):
    mesh = plsc.VectorSubcoreMesh(core_axis_name="c", subcore_axis_name="s", num_cores=NC, num_subcores=NS)
    f = pl.kernel(
        _scab2_body,
        out_type=[
            jax.ShapeDtypeStruct((NC, N, C2), jnp.float32),
            jax.ShapeDtypeStruct((NC, N), jnp.float32),
        ],
        mesh=mesh,
        compiler_params=pltpu.CompilerParams(use_tc_tiling_on_sc=False, needs_layout_passes=False),
        scratch_types=[
            pltpu.VMEM((_C2K,), jnp.int32),
            pltpu.VMEM((_C2K,), jnp.int32),
            pltpu.VMEM((_C2K, 8), jnp.float32),
            pltpu.VMEM((_C2K, 8), jnp.float32),
            pltpu.VMEM((_C2K, C2), jnp.float32),
            pltpu.VMEM((_C2K,), jnp.float32),
            pltpu.SemaphoreType.DMA,
            pltpu.VMEM_SHARED((N, C2), jnp.float32),
            pltpu.VMEM_SHARED((N,), jnp.float32),
        ],
    )
    return f(src, dst, sa2, h2)


_BN3 = 1000


def _tc3_body(out2p_ref, den2p_ref, h2_ref, sa2_ref, b2_ref, fcw_ref, fcb_ref,
              emb_ref, out_ref):
    sa_sum = sa2_ref[:, 0:1] + sa2_ref[:, 1:2]
    eself = jnp.exp(jnp.maximum(sa_sum, 0.2 * sa_sum))
    den = den2p_ref[:, 0:1] + den2p_ref[:, 1:2] + eself + 1e-16
    o = out2p_ref[0] + out2p_ref[1] + eself * h2_ref[...]
    emb = jnp.maximum(o / den + b2_ref[0, :], 0.0)
    emb_ref[...] = emb
    out_ref[...] = jnp.dot(emb, fcw_ref[...],
                           preferred_element_type=jnp.float32,
                 precision=lax.Precision.HIGHEST) + fcb_ref[0, :]


def _tc3(out2p, den2p, h2, sa2, b2, fc_W, fc_b):
    grid = (N // _BN3,)
    return pl.pallas_call(
        _tc3_body,
        grid=grid,
        in_specs=[
            pl.BlockSpec((NC, _BN3, C2), lambda i: (0, i, 0)),
            pl.BlockSpec((_BN3, NC), lambda i: (i, 0)),
            pl.BlockSpec((_BN3, C2), lambda i: (i, 0)),
            pl.BlockSpec((_BN3, 8), lambda i: (i, 0)),
            pl.BlockSpec((1, C2), lambda i: (0, 0)),
            pl.BlockSpec((C2, DOUT), lambda i: (0, 0)),
            pl.BlockSpec((1, DOUT), lambda i: (0, 0)),
        ],
        out_specs=[
            pl.BlockSpec((_BN3, C2), lambda i: (i, 0)),
            pl.BlockSpec((_BN3, DOUT), lambda i: (i, 0)),
        ],
        out_shape=[
            jax.ShapeDtypeStruct((N, C2), jnp.float32),
            jax.ShapeDtypeStruct((N, DOUT), jnp.float32),
        ],
    )(out2p, den2p, h2, sa2, b2, fc_W, fc_b)


def kernel(x, edge_index, W1, a_src1, a_dst1, b1, W2, a_src2, a_dst2, b2,
           fc_W, fc_b):
    src = edge_index[0]
    dst = edge_index[1]

    eye = jnp.eye(H1, dtype=jnp.float32)
    AsdA = (a_src1[:, :, None] * eye[:, None, :]).reshape(D1, H1)
    AsdB = (a_dst1[:, :, None] * eye[:, None, :]).reshape(D1, H1)
    Asd1 = jnp.concatenate([AsdA, AsdB], axis=1)
    Asd2 = jnp.concatenate(
        [a_src2[0][:, None], a_dst2[0][:, None],
         jnp.zeros((C2, 6), jnp.float32)], axis=1)

    h1t, saA, saB = _tc1(x, W1, Asd1)

    evT, den_p = _sca1(src, dst, saA, saB)
    h1f = h1t.reshape(H1 * N, C1)
    out1t = _scb1(src, dst, h1f, evT)
    h2, sa2 = _tc2(out1t, den_p, h1t, saA, saB, b1[None, :], W2, Asd2)

    out2p, den2p = _scab2(src, dst, sa2, h2)
    emb, out = _tc3(out2p, den2p.T, h2, sa2, b2[None, :], fc_W,
                    fc_b[None, :])
    return (emb, out)


# speedup vs baseline: 45.6566x; 1.2835x over previous
import functools

import jax
import jax.numpy as jnp
from jax import lax
from jax.experimental import pallas as pl
from jax.experimental.pallas import tpu as pltpu
from jax.experimental.pallas import tpu_sc as plsc

N = 10000
E = 320000
DIN = 128
H1 = 8
C1 = 64
D1 = H1 * C1
C2 = 32
DOUT = 64

NC = 2
NS = 16
NW = NC * NS


_BN1 = 1000


def _tc1_body(x_ref, w1_ref, asd_ref, h1t_ref, saA_ref, saB_ref):
    h = jnp.dot(x_ref[...], w1_ref[...], preferred_element_type=jnp.float32,
                 precision=lax.Precision.HIGHEST)
    sa = jnp.dot(h, asd_ref[...], preferred_element_type=jnp.float32,
                 precision=lax.Precision.HIGHEST)
    saA_ref[...] = sa[:, :H1]
    saB_ref[...] = sa[:, H1:]
    for p in range(H1 // 2):
        h1t_ref[p] = h[:, p * 2 * C1:(p + 1) * 2 * C1]


def _tc1(x, W1, Asd1):
    grid = (N // _BN1,)
    return pl.pallas_call(
        _tc1_body,
        grid=grid,
        in_specs=[
            pl.BlockSpec((_BN1, DIN), lambda i: (i, 0)),
            pl.BlockSpec((DIN, D1), lambda i: (0, 0)),
            pl.BlockSpec((D1, 2 * H1), lambda i: (0, 0)),
        ],
        out_specs=[
            pl.BlockSpec((H1 // 2, _BN1, 2 * C1), lambda i: (0, i, 0)),
            pl.BlockSpec((_BN1, H1), lambda i: (i, 0)),
            pl.BlockSpec((_BN1, H1), lambda i: (i, 0)),
        ],
        out_shape=[
            jax.ShapeDtypeStruct((H1 // 2, N, 2 * C1), jnp.float32),
            jax.ShapeDtypeStruct((N, H1), jnp.float32),
            jax.ShapeDtypeStruct((N, H1), jnp.float32),
        ],
    )(x, W1, Asd1)


_CA = 2000
_EPW = E // NW


def _sca1_body(src_hbm, dst_hbm, saA_hbm, saB_hbm, evT_hbm, den_hbm,
               idxS, idxD, rowsA, rowsB, ev2d, evT2d, sem, den_sp):
    c = lax.axis_index("c")
    s = lax.axis_index("s")
    wid = c * NS + s

    iota = lax.iota(jnp.int32, 16)
    ioh = iota // 8
    iol = iota % 8
    zeros16 = jnp.zeros((16,), jnp.float32)

    def _z(j, _):
        plsc.store_scatter(ev2d, [2 * j + ioh, iol], zeros16)
        return 0
    lax.fori_loop(0, _CA // 2, _z, 0, unroll=8)

    @pl.when(s < 5)
    def _():
        pltpu.sync_copy(ev2d, den_sp.at[pl.ds(s * _CA, _CA)])
    plsc.subcore_barrier()

    def chunk_body(k, _):
        base = wid * _EPW + k * _CA
        pltpu.sync_copy(src_hbm.at[pl.ds(base, _CA)], idxS)
        pltpu.sync_copy(dst_hbm.at[pl.ds(base, _CA)], idxD)
        cpA = pltpu.async_copy(saA_hbm.at[idxS], rowsA, sem)
        cpA.wait()
        cpB = pltpu.async_copy(saB_hbm.at[idxD], rowsB, sem)
        cpB.wait()

        def ebody(j, _):
            va = plsc.load_gather(rowsA, [2 * j + ioh, iol])
            vb = plsc.load_gather(rowsB, [2 * j + ioh, iol])
            lg = va + vb
            lg = jnp.maximum(lg, 0.2 * lg)
            e = jnp.exp(lg)
            plsc.store_scatter(ev2d, [2 * j + ioh, iol], e)
            return 0
        lax.fori_loop(0, _CA // 2, ebody, 0, unroll=8)

        def tbody(j, _):
            row = 16 * j + iota
            for hd in range(H1):
                v = plsc.load_gather(ev2d, [row, jnp.full((16,), hd, jnp.int32)])
                evT2d[hd, pl.ds(16 * j, 16)] = v
            return 0
        lax.fori_loop(0, _CA // 16, tbody, 0, unroll=2)
        for hd in range(H1):
            pltpu.sync_copy(evT2d.at[hd], evT_hbm.at[hd, pl.ds(base, _CA)])

        pltpu.sync_copy(ev2d, den_sp.at[idxD], add=True)
        return 0

    lax.fori_loop(0, _EPW // _CA, chunk_body, 0)

    plsc.subcore_barrier()
    @pl.when(s < 5)
    def _():
        pltpu.sync_copy(den_sp.at[pl.ds(s * _CA, _CA)],
                        den_hbm.at[c, pl.ds(s * _CA, _CA)])


def _sca1(src, dst, saA, saB):
    mesh = plsc.VectorSubcoreMesh(core_axis_name="c", subcore_axis_name="s", num_cores=NC, num_subcores=NS)
    f = pl.kernel(
        _sca1_body,
        out_type=[
            jax.ShapeDtypeStruct((H1, E), jnp.float32),
            jax.ShapeDtypeStruct((NC, N, H1), jnp.float32),
        ],
        mesh=mesh,
        compiler_params=pltpu.CompilerParams(use_tc_tiling_on_sc=False, needs_layout_passes=False),
        scratch_types=[
            pltpu.VMEM((_CA,), jnp.int32),
            pltpu.VMEM((_CA,), jnp.int32),
            pltpu.VMEM((_CA, H1), jnp.float32),
            pltpu.VMEM((_CA, H1), jnp.float32),
            pltpu.VMEM((_CA, H1), jnp.float32),
            pltpu.VMEM((H1, _CA), jnp.float32),
            pltpu.SemaphoreType.DMA,
            pltpu.VMEM_SHARED((N, H1), jnp.float32),
        ],
    )
    return f(src, dst, saA, saB)


_CB = 400
_EPS = E // NS
_NCK = _EPS // _CB
_ZROWS = N // NS


def _scb1_body(src_hbm, dst_hbm, h1f_hbm, evT_hbm, out_hbm,
               idxSa, idxDa, idxAa, rowsa, eva,
               idxSb, idxDb, idxAb, rowsb, evb,
               sema, semb, semi, acc_sp):
    c = lax.axis_index("c")
    s = lax.axis_index("s")

    for p in range(H1 // NC):
        hd = c * (H1 // NC) + p
        pair = hd // 2
        rowoff = 2 * pair * N + (hd % 2)

        def issue(k, idxS, idxD, idxA, rows, ev, sem):
            base = s * _EPS + k * _CB
            c1 = pltpu.async_copy(src_hbm.at[pl.ds(base, _CB)], idxS, semi)
            c2 = pltpu.async_copy(dst_hbm.at[pl.ds(base, _CB)], idxD, semi)
            c3 = pltpu.async_copy(evT_hbm.at[hd, pl.ds(base, _CB)], ev, semi)
            c1.wait(); c2.wait(); c3.wait()

            def abody(j, _):
                idxA[pl.ds(16 * j, 16)] = idxS[pl.ds(16 * j, 16)] * 2 + rowoff
                return 0
            lax.fori_loop(0, _CB // 16, abody, 0, unroll=8)
            pltpu.async_copy(h1f_hbm.at[idxA], rows, sem)

        def drain(idxA, rows, sem):
            pltpu.make_async_copy(h1f_hbm.at[idxA], rows, sem).wait()

        def scale_scatter(idxD, rows, ev):
            @plsc.parallel_loop(0, _CB, unroll=8)
            def sbody(b):
                w = plsc.load_gather(ev, [jnp.full((16,), b, jnp.int32)])
                for q in range(C1 // 16):
                    rows[b, pl.ds(16 * q, 16)] = rows[b, pl.ds(16 * q, 16)] * w
            pltpu.sync_copy(rows, acc_sp.at[idxD], add=True)

        def _z(j, _):
            for q in range(C1 // 16):
                rowsa[j, pl.ds(16 * q, 16)] = jnp.zeros((16,), jnp.float32)
            return 0
        lax.fori_loop(0, _CB, _z, 0, unroll=8)

        def zcopy(j, _):
            pltpu.sync_copy(rowsa,
                            acc_sp.at[pl.ds(s * _ZROWS + j * _CB, _CB)])
            return 0
        lax.fori_loop(0, _ZROWS // _CB, zcopy, 0)
        pltpu.sync_copy(rowsa.at[pl.ds(0, _ZROWS - _ZROWS // _CB * _CB)],
                        acc_sp.at[pl.ds(s * _ZROWS + _ZROWS // _CB * _CB,
                                        _ZROWS - _ZROWS // _CB * _CB)])
        plsc.subcore_barrier()

        issue(0, idxSa, idxDa, idxAa, rowsa, eva, sema)

        def body2(t, _):
            k0 = 2 * t
            drain(idxAa, rowsa, sema)
            issue(k0 + 1, idxSb, idxDb, idxAb, rowsb, evb, semb)
            scale_scatter(idxDa, rowsa, eva)
            drain(idxAb, rowsb, semb)
            @pl.when(t < _NCK // 2 - 1)
            def _():
                issue(k0 + 2, idxSa, idxDa, idxAa, rowsa, eva, sema)
            scale_scatter(idxDb, rowsb, evb)
            return 0

        lax.fori_loop(0, _NCK // 2, body2, 0)

        plsc.subcore_barrier()
        pltpu.sync_copy(acc_sp.at[pl.ds(s * _ZROWS, _ZROWS)],
                        out_hbm.at[hd, pl.ds(s * _ZROWS, _ZROWS)])
        plsc.subcore_barrier()


def _scb1(src, dst, h1f, evT):
    mesh = plsc.VectorSubcoreMesh(core_axis_name="c", subcore_axis_name="s", num_cores=NC, num_subcores=NS)
    f = pl.kernel(
        _scb1_body,
        out_type=[jax.ShapeDtypeStruct((H1, N, C1), jnp.float32)],
        mesh=mesh,
        compiler_params=pltpu.CompilerParams(use_tc_tiling_on_sc=False, needs_layout_passes=False),
        scratch_types=[
            pltpu.VMEM((_CB,), jnp.int32),
            pltpu.VMEM((_CB,), jnp.int32),
            pltpu.VMEM((_CB,), jnp.int32),
            pltpu.VMEM((_CB, C1), jnp.float32),
            pltpu.VMEM((_CB,), jnp.float32),
            pltpu.VMEM((_CB,), jnp.int32),
            pltpu.VMEM((_CB,), jnp.int32),
            pltpu.VMEM((_CB,), jnp.int32),
            pltpu.VMEM((_CB, C1), jnp.float32),
            pltpu.VMEM((_CB,), jnp.float32),
            pltpu.SemaphoreType.DMA,
            pltpu.SemaphoreType.DMA,
            pltpu.SemaphoreType.DMA,
            pltpu.VMEM_SHARED((N, C1), jnp.float32),
        ],
    )
    return f(src, dst, h1f, evT)[0]


_BN2 = 1000


def _tc2_body(out1t_ref, denp_ref, h1t_ref, saA_ref, saB_ref, b1_ref,
              w2_ref, asd2_ref, h2_ref, sa2_ref):
    sa_sum = saA_ref[...] + saB_ref[...]
    eself = jnp.exp(jnp.maximum(sa_sum, 0.2 * sa_sum))
    den = denp_ref[0] + denp_ref[1] + eself + 1e-16
    parts = []
    for hd in range(H1):
        p, q = hd // 2, (hd % 2) * C1
        o = (out1t_ref[hd]
             + eself[:, hd:hd + 1] * h1t_ref[p][:, q:q + C1])
        o = o / den[:, hd:hd + 1] + b1_ref[0, hd * C1:(hd + 1) * C1]
        parts.append(jnp.maximum(o, 0.0))
    act = jnp.concatenate(parts, axis=1)
    h2 = jnp.dot(act, w2_ref[...], preferred_element_type=jnp.float32,
                 precision=lax.Precision.HIGHEST)
    h2_ref[...] = h2
    sa2_ref[...] = jnp.dot(h2, asd2_ref[...], preferred_element_type=jnp.float32,
                 precision=lax.Precision.HIGHEST)


def _tc2(out1t, den_p, h1t, saA, saB, b1, W2, Asd2):
    grid = (N // _BN2,)
    return pl.pallas_call(
        _tc2_body,
        grid=grid,
        in_specs=[
            pl.BlockSpec((H1, _BN2, C1), lambda i: (0, i, 0)),
            pl.BlockSpec((NC, _BN2, H1), lambda i: (0, i, 0)),
            pl.BlockSpec((H1 // 2, _BN2, 2 * C1), lambda i: (0, i, 0)),
            pl.BlockSpec((_BN2, H1), lambda i: (i, 0)),
            pl.BlockSpec((_BN2, H1), lambda i: (i, 0)),
            pl.BlockSpec((1, D1), lambda i: (0, 0)),
            pl.BlockSpec((D1, C2), lambda i: (0, 0)),
            pl.BlockSpec((C2, 8), lambda i: (0, 0)),
        ],
        out_specs=[
            pl.BlockSpec((_BN2, C2), lambda i: (i, 0)),
            pl.BlockSpec((_BN2, 8), lambda i: (i, 0)),
        ],
        out_shape=[
            jax.ShapeDtypeStruct((N, C2), jnp.float32),
            jax.ShapeDtypeStruct((N, 8), jnp.float32),
        ],
    )(out1t, den_p, h1t, saA, saB, b1, W2, Asd2)


_C2K = 400
_ZR2 = 640


def _scab2_body(src_hbm, dst_hbm, sa2_hbm, h2_hbm, out_hbm, den_hbm,
                idxS, idxD, rowsS, rowsD, rows, ev, sem, acc_sp, den_sp):
    c = lax.axis_index("c")
    s = lax.axis_index("s")
    wid = c * NS + s
    iota = lax.iota(jnp.int32, 16)

    if True:
        def _z(j, _):
            for q in range(C2 // 16):
                rows[j, pl.ds(16 * q, 16)] = jnp.zeros((16,), jnp.float32)
            return 0
        lax.fori_loop(0, _ZR2, _z, 0, unroll=8)
        def _z2(j, _):
            ev[pl.ds(16 * j, 16)] = jnp.zeros((16,), jnp.float32)
            return 0
        lax.fori_loop(0, _ZR2 // 16, _z2, 0, unroll=8)

        pltpu.sync_copy(rows.at[pl.ds(0, _ZR2)],
                        acc_sp.at[pl.ds(s * _ZR2, _ZR2)])
        pltpu.sync_copy(ev.at[pl.ds(0, _ZR2)], den_sp.at[pl.ds(s * _ZR2, _ZR2)])
        plsc.subcore_barrier()

        def chunk_body(k, _):
            base = wid * _EPW + k * _C2K
            pltpu.sync_copy(src_hbm.at[pl.ds(base, _C2K)], idxS)
            pltpu.sync_copy(dst_hbm.at[pl.ds(base, _C2K)], idxD)
            pltpu.async_copy(sa2_hbm.at[idxS], rowsS, sem).wait()
            pltpu.async_copy(sa2_hbm.at[idxD], rowsD, sem).wait()
            pltpu.async_copy(h2_hbm.at[idxS], rows, sem).wait()

            def ebody(j, _):
                row = 16 * j + iota
                va = plsc.load_gather(rowsS, [row, jnp.zeros((16,), jnp.int32)])
                vb = plsc.load_gather(rowsD, [row, jnp.ones((16,), jnp.int32)])
                lg = va + vb
                lg = jnp.maximum(lg, 0.2 * lg)
                ev[pl.ds(16 * j, 16)] = jnp.exp(lg)
                return 0
            lax.fori_loop(0, _C2K // 16, ebody, 0, unroll=4)

            pltpu.sync_copy(ev, den_sp.at[idxD], add=True)

            @plsc.parallel_loop(0, _C2K, unroll=8)
            def sbody(b):
                w = plsc.load_gather(ev, [jnp.full((16,), b, jnp.int32)])
                rows[b, pl.ds(0, 16)] = rows[b, pl.ds(0, 16)] * w
                rows[b, pl.ds(16, 16)] = rows[b, pl.ds(16, 16)] * w

            pltpu.sync_copy(rows, acc_sp.at[idxD], add=True)
            return 0

        lax.fori_loop(0, _EPW // _C2K, chunk_body, 0)

        plsc.subcore_barrier()
        base = s * _ZR2
        @pl.when(s < NS - 1)
        def _():
            pltpu.sync_copy(acc_sp.at[pl.ds(base, _ZR2)],
                            out_hbm.at[c, pl.ds(base, _ZR2)])
            pltpu.sync_copy(den_sp.at[pl.ds(base, _ZR2)],
                            den_hbm.at[c, pl.ds(base, _ZR2)])
        @pl.when(s == NS - 1)
        def _():
            last = N - _ZR2 * (NS - 1)
            pltpu.sync_copy(acc_sp.at[pl.ds(base, last)],
                            out_hbm.at[c, pl.ds(base, last)])
            pltpu.sync_copy(den_sp.at[pl.ds(base, last)],
                            den_hbm.at[c, pl.ds(base, last)])


def _scab2(src, dst, sa2, h2):
    mesh = plsc.VectorSubcoreMesh(core_axis_name="c", subcore_axis_name="s", num_cores=NC, num_subcores=NS)
    f = pl.kernel(
        _scab2_body,
        out_type=[
            jax.ShapeDtypeStruct((NC, N, C2), jnp.float32),
            jax.ShapeDtypeStruct((NC, N), jnp.float32),
        ],
        mesh=mesh,
        compiler_params=pltpu.CompilerParams(use_tc_tiling_on_sc=False, needs_layout_passes=False),
        scratch_types=[
            pltpu.VMEM((_C2K,), jnp.int32),
            pltpu.VMEM((_C2K,), jnp.int32),
            pltpu.VMEM((_C2K, 8), jnp.float32),
            pltpu.VMEM((_C2K, 8), jnp.float32),
            pltpu.VMEM((_C2K, C2), jnp.float32),
            pltpu.VMEM((_C2K,), jnp.float32),
            pltpu.SemaphoreType.DMA,
            pltpu.VMEM_SHARED((N, C2), jnp.float32),
            pltpu.VMEM_SHARED((N,), jnp.float32),
        ],
    )
    return f(src, dst, sa2, h2)


_BN3 = 1000


def _tc3_body(out2p_ref, den2p_ref, h2_ref, sa2_ref, b2_ref, fcw_ref, fcb_ref,
              emb_ref, out_ref):
    sa_sum = sa2_ref[:, 0:1] + sa2_ref[:, 1:2]
    eself = jnp.exp(jnp.maximum(sa_sum, 0.2 * sa_sum))
    den = den2p_ref[:, 0:1] + den2p_ref[:, 1:2] + eself + 1e-16
    o = out2p_ref[0] + out2p_ref[1] + eself * h2_ref[...]
    emb = jnp.maximum(o / den + b2_ref[0, :], 0.0)
    emb_ref[...] = emb
    out_ref[...] = jnp.dot(emb, fcw_ref[...],
                           preferred_element_type=jnp.float32,
                 precision=lax.Precision.HIGHEST) + fcb_ref[0, :]


def _tc3(out2p, den2p, h2, sa2, b2, fc_W, fc_b):
    grid = (N // _BN3,)
    return pl.pallas_call(
        _tc3_body,
        grid=grid,
        in_specs=[
            pl.BlockSpec((NC, _BN3, C2), lambda i: (0, i, 0)),
            pl.BlockSpec((_BN3, NC), lambda i: (i, 0)),
            pl.BlockSpec((_BN3, C2), lambda i: (i, 0)),
            pl.BlockSpec((_BN3, 8), lambda i: (i, 0)),
            pl.BlockSpec((1, C2), lambda i: (0, 0)),
            pl.BlockSpec((C2, DOUT), lambda i: (0, 0)),
            pl.BlockSpec((1, DOUT), lambda i: (0, 0)),
        ],
        out_specs=[
            pl.BlockSpec((_BN3, C2), lambda i: (i, 0)),
            pl.BlockSpec((_BN3, DOUT), lambda i: (i, 0)),
        ],
        out_shape=[
            jax.ShapeDtypeStruct((N, C2), jnp.float32),
            jax.ShapeDtypeStruct((N, DOUT), jnp.float32),
        ],
    )(out2p, den2p, h2, sa2, b2, fc_W, fc_b)


def kernel(x, edge_index, W1, a_src1, a_dst1, b1, W2, a_src2, a_dst2, b2,
           fc_W, fc_b):
    src = edge_index[0]
    dst = edge_index[1]

    eye = jnp.eye(H1, dtype=jnp.float32)
    AsdA = (a_src1[:, :, None] * eye[:, None, :]).reshape(D1, H1)
    AsdB = (a_dst1[:, :, None] * eye[:, None, :]).reshape(D1, H1)
    Asd1 = jnp.concatenate([AsdA, AsdB], axis=1)
    Asd2 = jnp.concatenate(
        [a_src2[0][:, None], a_dst2[0][:, None],
         jnp.zeros((C2, 6), jnp.float32)], axis=1)

    h1t, saA, saB = _tc1(x, W1, Asd1)

    evT, den_p = _sca1(src, dst, saA, saB)
    h1f = h1t.reshape(H1 * N, C1)
    out1t = _scb1(src, dst, h1f, evT)
    h2, sa2 = _tc2(out1t, den_p, h1t, saA, saB, b1[None, :], W2, Asd2)

    out2p, den2p = _scab2(src, dst, sa2, h2)
    emb, out = _tc3(out2p, den2p.T, h2, sa2, b2[None, :], fc_W,
                    fc_b[None, :])
    return (emb, out)
